# baseline (speedup 1.0000x reference)
import functools

import jax
import jax.numpy as jnp
from jax import lax
from jax.experimental import pallas as pl
from jax.experimental.pallas import tpu as pltpu
from jax.experimental.pallas import tpu_sc as plsc

N = 10000
E = 320000
D = 128
H = 128
C = 40
EPS = 1e-5

NC = 2
NS = 16
NW = NC * NS
K = 128
T = 84
E_PAD = NW * T * K
NCHUNK = E_PAD // K
OROWS = 628
RALL = NS * OROWS
ACC_ROWS = RALL

NBUF = 3
IBX = 4
UNROLL = 12
G = T // UNROLL


@functools.lru_cache(maxsize=None)
def _sc_kernels():
    mesh = plsc.VectorSubcoreMesh(core_axis_name="c", subcore_axis_name="s",
                                  num_cores=NC, num_subcores=NS)

    @functools.partial(
        pl.kernel,
        out_type=jax.ShapeDtypeStruct((NC, NS, OROWS, H), jnp.float32),
        mesh=mesh,
        scratch_types=[
            pltpu.VMEM((T, K), jnp.int32),
            pltpu.VMEM((K, H), jnp.float32),
            pltpu.VMEM_SHARED((ACC_ROWS, H), jnp.float32),
            pltpu.SemaphoreType.DMA,
        ],
    )
    def deg_kernel(dst_hbm, ones_hbm, zeros_hbm, out_hbm, dst_all, ones_v,
                   acc, sem):
        cid = lax.axis_index("c")
        sid = lax.axis_index("s")
        wid = sid * NC + cid
        pltpu.sync_copy(zeros_hbm, acc.at[pl.ds(sid * OROWS, OROWS)])
        pltpu.sync_copy(ones_hbm, ones_v)
        pltpu.sync_copy(dst_hbm.at[wid], dst_all)
        plsc.subcore_barrier()

        def chunk(c, carry):
            pltpu.async_copy(ones_v, acc.at[dst_all.at[c]], sem, add=True)
            return carry

        lax.fori_loop(0, T, chunk, 0)

        def drain(c, carry):
            pltpu.make_async_copy(ones_v, acc.at[dst_all.at[c]], sem).wait()
            return carry

        lax.fori_loop(0, T, drain, 0)
        plsc.subcore_barrier()
        pltpu.sync_copy(acc.at[pl.ds(sid * OROWS, OROWS)],
                        out_hbm.at[cid].at[sid])

    @functools.partial(
        pl.kernel,
        out_type=jax.ShapeDtypeStruct((NC, NS, OROWS, H), jnp.float32),
        mesh=mesh,
        scratch_types=[
            pltpu.VMEM((IBX, 2, K), jnp.int32),
            pltpu.VMEM((NBUF, K, H), jnp.float32),
            pltpu.VMEM_SHARED((ACC_ROWS, H), jnp.float32),
            pltpu.SemaphoreType.DMA((NBUF,)),
            pltpu.SemaphoreType.DMA((NBUF,)),
            pltpu.SemaphoreType.DMA((IBX,)),
        ],
    )
    def agg_kernel(h_hbm, sd_hbm, zeros_hbm, out_hbm,
                   cidx, rows, acc, gsem, ssem, isem):
        cid = lax.axis_index("c")
        sid = lax.axis_index("s")
        wid = sid * NC + cid
        base = wid * T
        pltpu.sync_copy(zeros_hbm, acc.at[pl.ds(sid * OROWS, OROWS)])
        plsc.subcore_barrier()

        def fetch_idx(c, slot):
            pltpu.async_copy(sd_hbm.at[base + c], cidx.at[slot],
                             isem.at[slot])

        def wait_idx(c, slot):
            pltpu.make_async_copy(sd_hbm.at[base + c], cidx.at[slot],
                                  isem.at[slot]).wait()

        def start_gather(slot, b):
            pltpu.async_copy(h_hbm.at[cidx.at[slot].at[0]], rows.at[b],
                             gsem.at[b])

        def wait_gather(slot, b):
            pltpu.make_async_copy(h_hbm.at[cidx.at[slot].at[0]], rows.at[b],
                                  gsem.at[b]).wait()

        def start_scatter(slot, b):
            pltpu.async_copy(rows.at[b], acc.at[cidx.at[slot].at[1]],
                             ssem.at[b], add=True)

        def wait_scatter(slot, b):
            pltpu.make_async_copy(rows.at[b], acc.at[cidx.at[slot].at[1]],
                                  ssem.at[b]).wait()

        for j in range(3):
            fetch_idx(j, j)
        for j in range(2):
            wait_idx(j, j)
            start_gather(j, j)

        def group(g, carry):
            for u in range(UNROLL):
                c = g * UNROLL + u
                b = u % NBUF
                i = u % IBX
                wait_gather(i, b)
                start_scatter(i, b)
                ws = lambda: wait_scatter((u + 3) % IBX, (u + 2) % NBUF)
                if u == 0:
                    pl.when(g > 0)(ws)
                else:
                    ws()
                sg = lambda: start_gather((u + 2) % IBX, (u + 2) % NBUF)
                if u < UNROLL - 2:
                    wait_idx(c + 2, (u + 2) % IBX)
                    sg()
                else:
                    def wg():
                        wait_idx(c + 2, (u + 2) % IBX)
                        sg()
                    pl.when(g < G - 1)(wg)
                pf = lambda: fetch_idx(c + 3, (u + 3) % IBX)
                if u < UNROLL - 3:
                    pf()
                else:
                    pl.when(g < G - 1)(pf)
            return carry

        lax.fori_loop(0, G, group, 0)
        wait_scatter((UNROLL - 1) % IBX, (UNROLL - 1) % NBUF)
        plsc.subcore_barrier()
        pltpu.sync_copy(acc.at[pl.ds(sid * OROWS, OROWS)],
                        out_hbm.at[cid].at[sid])

    return deg_kernel, agg_kernel


BR = 1256


def _dinv(d0_ref, d1_ref):
    deg = d0_ref[:, 0:1] + d1_ref[:, 0:1] + 1.0
    return lax.rsqrt(deg)


def _a1_body(x_ref, w_ref, d0_ref, d1_ref, o_ref):
    h = jnp.dot(x_ref[...], w_ref[...], preferred_element_type=jnp.float32)
    o_ref[...] = h * _dinv(d0_ref, d1_ref)


def _bn_relu(p0, p1, hp, dinv, b, g, beta, rm, rv):
    agg = (p0[...] + p1[...] + hp[...]) * dinv + b[...]
    y = (agg - rm[...]) * lax.rsqrt(rv[...] + EPS) * g[...] + beta[...]
    return jnp.maximum(y, 0.0)


def _b1a2_body(p0, p1, hp, d0, d1, b, g, beta, rm, rv, w2, o_ref):
    dinv = _dinv(d0, d1)
    y = _bn_relu(p0, p1, hp, dinv, b, g, beta, rm, rv)
    h2 = jnp.dot(y, w2[...], preferred_element_type=jnp.float32) * dinv
    row = pl.program_id(0) * BR + lax.broadcasted_iota(jnp.int32, h2.shape, 0)
    o_ref[...] = jnp.where(row < N, h2, 0.0)


def _b2fc_body(p0, p1, hp, d0, d1, b, g, beta, rm, rv, fw, fb, o_ref):
    dinv = _dinv(d0, d1)
    y = _bn_relu(p0, p1, hp, dinv, b, g, beta, rm, rv)
    logits = jnp.dot(y, fw[...], preferred_element_type=jnp.float32) + fb[...]
    col = lax.broadcasted_iota(jnp.int32, logits.shape, 1)
    logits = jnp.where(col < C, logits, -1e30)
    m = jnp.max(logits, axis=1, keepdims=True)
    s = jnp.sum(jnp.exp(logits - m), axis=1, keepdims=True)
    o_ref[...] = logits - m - jnp.log(s)


def _row_spec(w):
    return pl.BlockSpec((BR, w), lambda i: (i, 0))


def _full_spec(r, c):
    return pl.BlockSpec((r, c), lambda i: (0, 0))


_NH = jax.ShapeDtypeStruct((RALL, H), jnp.float32)

_a1_call = pl.pallas_call(
    _a1_body, grid=(RALL // BR,),
    in_specs=[_row_spec(D), _full_spec(D, H), _row_spec(H), _row_spec(H)],
    out_specs=_row_spec(H), out_shape=_NH)

_b1a2_call = pl.pallas_call(
    _b1a2_body, grid=(RALL // BR,),
    in_specs=[_row_spec(H), _row_spec(H), _row_spec(H), _row_spec(H),
              _row_spec(H)] + [_full_spec(1, H)] * 5 + [_full_spec(H, H)],
    out_specs=_row_spec(H), out_shape=_NH)

_b2fc_call = pl.pallas_call(
    _b2fc_body, grid=(RALL // BR,),
    in_specs=[_row_spec(H), _row_spec(H), _row_spec(H), _row_spec(H),
              _row_spec(H)] + [_full_spec(1, H)] * 5
             + [_full_spec(H, H), _full_spec(1, H)],
    out_specs=_row_spec(H), out_shape=_NH)


def kernel(x, edge_index, W1, b1, g1, beta1, rm1, rv1,
           W2, b2, g2, beta2, rm2, rv2, fcW, fcb):
    src = edge_index[0].astype(jnp.int32)
    dst = edge_index[1].astype(jnp.int32)
    pad = E_PAD - E
    src_p = jnp.concatenate([src, jnp.full((pad,), N, jnp.int32)])
    dst_p = jnp.concatenate([dst, jnp.zeros((pad,), jnp.int32)])
    sd = jnp.stack([src_p.reshape(NCHUNK, K), dst_p.reshape(NCHUNK, K)], 1)
    dst_deg = jnp.concatenate(
        [dst, jnp.full((pad,), N, jnp.int32)]).reshape(NW, T, K)

    onesH = jnp.ones((K, H), jnp.float32)
    zerosH = jnp.zeros((OROWS, H), jnp.float32)
    x_pad = jnp.pad(x, ((0, RALL - N), (0, 0)))

    deg_kernel, agg_kernel = _sc_kernels()
    degp = deg_kernel(dst_deg, onesH, zerosH)
    d0 = degp[0].reshape(RALL, H)
    d1 = degp[1].reshape(RALL, H)

    b1r, g1r, be1r = b1.reshape(1, H), g1.reshape(1, H), beta1.reshape(1, H)
    rm1r, rv1r = rm1.reshape(1, H), rv1.reshape(1, H)
    b2r, g2r, be2r = b2.reshape(1, H), g2.reshape(1, H), beta2.reshape(1, H)
    rm2r, rv2r = rm2.reshape(1, H), rv2.reshape(1, H)
    fw = jnp.pad(fcW, ((0, 0), (0, H - C)))
    fb = jnp.pad(fcb, (0, H - C)).reshape(1, H)

    h1p = _a1_call(x_pad, W1, d0, d1)
    p1 = agg_kernel(h1p, sd, zerosH)
    h2p = _b1a2_call(p1[0].reshape(RALL, H), p1[1].reshape(RALL, H), h1p,
                     d0, d1, b1r, g1r, be1r, rm1r, rv1r, W2)
    p2 = agg_kernel(h2p, sd, zerosH)
    out = _b2fc_call(p2[0].reshape(RALL, H), p2[1].reshape(RALL, H), h2p,
                     d0, d1, b2r, g2r, be2r, rm2r, rv2r, fw, fb)
    return out[:N, :C]


# speedup vs baseline: 1.9581x; 1.9581x over previous
import functools

import jax
import jax.numpy as jnp
from jax import lax
from jax.experimental import pallas as pl
from jax.experimental.pallas import tpu as pltpu
from jax.experimental.pallas import tpu_sc as plsc

N = 10000
E = 320000
D = 128
H = 128
C = 40
EPS = 1e-5

NC = 2
NS = 16
NW = NC * NS
K = 128
T = 80
E_PAD = NW * T * K
OROWS = 632
OPAD = NS * OROWS
ACC_ROWS = OPAD
ZROWS = OROWS

@functools.lru_cache(maxsize=None)
def _sc_kernels():
    mesh = plsc.VectorSubcoreMesh(core_axis_name="c", subcore_axis_name="s",
                                  num_cores=NC, num_subcores=NS)

    @functools.partial(
        pl.kernel,
        out_type=jax.ShapeDtypeStruct((NC, OPAD, H), jnp.float32),
        mesh=mesh,
        scratch_types=[
            pltpu.VMEM((T, K), jnp.int32),
            pltpu.VMEM((K, H), jnp.float32),
            pltpu.VMEM_SHARED((ACC_ROWS, H), jnp.float32),
            pltpu.SemaphoreType.DMA,
        ],
    )
    def deg_kernel(dst_hbm, ones_hbm, zeros_hbm, out_hbm, dst_all, ones_v,
                   acc, sem):
        cid = lax.axis_index("c")
        sid = lax.axis_index("s")
        wid = sid * NC + cid
        pltpu.sync_copy(zeros_hbm, acc.at[pl.ds(sid * ZROWS, ZROWS)])
        pltpu.sync_copy(ones_hbm, ones_v)
        pltpu.sync_copy(dst_hbm.at[pl.ds(wid * T, T)], dst_all)
        plsc.subcore_barrier()

        def chunk(c, carry):
            pltpu.async_copy(ones_v, acc.at[dst_all.at[c]], sem, add=True)
            return carry

        lax.fori_loop(0, T, chunk, 0)

        def drain(c, carry):
            pltpu.make_async_copy(ones_v, acc.at[dst_all.at[c]], sem).wait()
            return carry

        lax.fori_loop(0, T, drain, 0)
        plsc.subcore_barrier()
        pltpu.sync_copy(acc.at[pl.ds(sid * OROWS, OROWS)],
                        out_hbm.at[cid].at[pl.ds(sid * OROWS, OROWS)])

    NBUF = 2
    IB = 4
    G2 = T // IB

    @functools.partial(
        pl.kernel,
        out_type=jax.ShapeDtypeStruct((NC, OPAD, H), jnp.float32),
        mesh=mesh,
        scratch_types=[
            pltpu.VMEM((IB, K), jnp.int32),
            pltpu.VMEM((IB, K), jnp.int32),
            pltpu.VMEM((NBUF, K, H), jnp.float32),
            pltpu.VMEM_SHARED((ACC_ROWS, H), jnp.float32),
            pltpu.SemaphoreType.DMA((NBUF,)),
            pltpu.SemaphoreType.DMA((NBUF,)),
            pltpu.SemaphoreType.DMA((IB,)),
        ],
    )
    def agg_kernel(h_hbm, src_hbm, dst_hbm, zeros_hbm, out_hbm,
                   sidx, didx, rows, acc, gsem, ssem, isem):
        cid = lax.axis_index("c")
        sid = lax.axis_index("s")
        wid = sid * NC + cid
        base = wid * T
        pltpu.sync_copy(zeros_hbm, acc.at[pl.ds(sid * ZROWS, ZROWS)])
        for b in range(NBUF):
            pltpu.async_copy(src_hbm.at[base + b], sidx.at[b], isem.at[b])
            pltpu.async_copy(dst_hbm.at[base + b], didx.at[b], isem.at[b])
        plsc.subcore_barrier()

        def do_chunk(g, u, drain_only):
            b = u % NBUF
            c = g * IB + u
            pu = (u - NBUF) % IB

            def wait_scatter():
                pltpu.make_async_copy(rows.at[b], acc.at[didx.at[pu]],
                                      ssem.at[b]).wait()

            if u < NBUF:
                pl.when(g > 0)(wait_scatter)
            else:
                wait_scatter()
            if drain_only:
                return
            pf = (u + NBUF) % IB

            def prefetch():
                pltpu.async_copy(src_hbm.at[base + c + NBUF], sidx.at[pf],
                                 isem.at[pf])
                pltpu.async_copy(dst_hbm.at[base + c + NBUF], didx.at[pf],
                                 isem.at[pf])

            if u < NBUF:
                prefetch()
            else:
                pl.when(g < G2 - 1)(prefetch)
            pltpu.make_async_copy(src_hbm.at[base + c], sidx.at[u],
                                  isem.at[u]).wait()
            pltpu.make_async_copy(dst_hbm.at[base + c], didx.at[u],
                                  isem.at[u]).wait()
            h0, h1 = pl.ds(0, K // 2), pl.ds(K // 2, K // 2)
            for hs in (h0, h1):
                pltpu.async_copy(h_hbm.at[sidx.at[u].at[hs]],
                                 rows.at[b].at[hs], gsem.at[b])
            for hs in (h0, h1):
                pltpu.make_async_copy(h_hbm.at[sidx.at[u].at[hs]],
                                      rows.at[b].at[hs], gsem.at[b]).wait()
            pltpu.async_copy(rows.at[b], acc.at[didx.at[u]], ssem.at[b],
                             add=True)

        def group(g, carry):
            for u in range(IB):
                do_chunk(g, u, False)
            return carry

        lax.fori_loop(0, G2, group, 0)
        for u in range(NBUF, IB):
            pltpu.make_async_copy(rows.at[u % NBUF], acc.at[didx.at[u]],
                                  ssem.at[u % NBUF]).wait()
        plsc.subcore_barrier()
        pltpu.sync_copy(acc.at[pl.ds(sid * OROWS, OROWS)],
                        out_hbm.at[cid].at[pl.ds(sid * OROWS, OROWS)])

    return deg_kernel, agg_kernel


BR = 1000


def _dinv(d0_ref, d1_ref):
    deg = d0_ref[:, 0:1] + d1_ref[:, 0:1] + 1.0
    return lax.rsqrt(deg)


def _a1_body(x_ref, w_ref, d0_ref, d1_ref, o_ref):
    h = jnp.dot(x_ref[...], w_ref[...], preferred_element_type=jnp.float32)
    o_ref[...] = h * _dinv(d0_ref, d1_ref)


def _bn_relu(p0, p1, hp, dinv, b, g, beta, rm, rv):
    agg = (p0[...] + p1[...] + hp[...]) * dinv + b[...]
    y = (agg - rm[...]) * lax.rsqrt(rv[...] + EPS) * g[...] + beta[...]
    return jnp.maximum(y, 0.0)


def _b1a2_body(p0, p1, hp, d0, d1, b, g, beta, rm, rv, w2, o_ref):
    dinv = _dinv(d0, d1)
    y = _bn_relu(p0, p1, hp, dinv, b, g, beta, rm, rv)
    o_ref[...] = jnp.dot(y, w2[...], preferred_element_type=jnp.float32) * dinv


def _b2fc_body(p0, p1, hp, d0, d1, b, g, beta, rm, rv, fw, fb, o_ref):
    dinv = _dinv(d0, d1)
    y = _bn_relu(p0, p1, hp, dinv, b, g, beta, rm, rv)
    logits = jnp.dot(y, fw[...], preferred_element_type=jnp.float32) + fb[...]
    col = lax.broadcasted_iota(jnp.int32, logits.shape, 1)
    logits = jnp.where(col < C, logits, -1e30)
    m = jnp.max(logits, axis=1, keepdims=True)
    s = jnp.sum(jnp.exp(logits - m), axis=1, keepdims=True)
    o_ref[...] = logits - m - jnp.log(s)


def _row_spec(w):
    return pl.BlockSpec((BR, w), lambda i: (i, 0))


def _full_spec(r, c):
    return pl.BlockSpec((r, c), lambda i: (0, 0))


_NH = jax.ShapeDtypeStruct((N, H), jnp.float32)

_a1_call = pl.pallas_call(
    _a1_body, grid=(N // BR,),
    in_specs=[_row_spec(D), _full_spec(D, H), _row_spec(16), _row_spec(16)],
    out_specs=_row_spec(H), out_shape=_NH)

_b1a2_call = pl.pallas_call(
    _b1a2_body, grid=(N // BR,),
    in_specs=[_row_spec(H), _row_spec(H), _row_spec(H), _row_spec(16),
              _row_spec(16)] + [_full_spec(1, H)] * 5 + [_full_spec(H, H)],
    out_specs=_row_spec(H), out_shape=_NH)

_b2fc_call = pl.pallas_call(
    _b2fc_body, grid=(N // BR,),
    in_specs=[_row_spec(H), _row_spec(H), _row_spec(H), _row_spec(16),
              _row_spec(16)] + [_full_spec(1, H)] * 5
             + [_full_spec(H, H), _full_spec(1, H)],
    out_specs=_row_spec(H), out_shape=_NH)


def kernel(x, edge_index, W1, b1, g1, beta1, rm1, rv1,
           W2, b2, g2, beta2, rm2, rv2, fcW, fcb):
    src = edge_index[0].astype(jnp.int32)
    dst = edge_index[1].astype(jnp.int32)
    pad = E_PAD - E
    src_p = jnp.concatenate([src, jnp.zeros((pad,), jnp.int32)]).reshape(-1, K)
    dst_p = jnp.concatenate([dst, jnp.full((pad,), N, jnp.int32)]).reshape(-1, K)

    onesH = jnp.ones((K, H), jnp.float32)
    zerosH = jnp.zeros((ZROWS, H), jnp.float32)

    deg_kernel, agg_kernel = _sc_kernels()
    degp = deg_kernel(dst_p, onesH, zerosH)
    d0, d1 = degp[0, :N, :16], degp[1, :N, :16]

    b1r, g1r, be1r = b1.reshape(1, H), g1.reshape(1, H), beta1.reshape(1, H)
    rm1r, rv1r = rm1.reshape(1, H), rv1.reshape(1, H)
    b2r, g2r, be2r = b2.reshape(1, H), g2.reshape(1, H), beta2.reshape(1, H)
    rm2r, rv2r = rm2.reshape(1, H), rv2.reshape(1, H)
    fw = jnp.pad(fcW, ((0, 0), (0, H - C)))
    fb = jnp.pad(fcb, (0, H - C)).reshape(1, H)

    h1p = _a1_call(x, W1, d0, d1)
    p1 = agg_kernel(h1p, src_p, dst_p, zerosH)
    h2p = _b1a2_call(p1[0, :N], p1[1, :N], h1p, d0, d1,
                     b1r, g1r, be1r, rm1r, rv1r, W2)
    p2 = agg_kernel(h2p, src_p, dst_p, zerosH)
    out = _b2fc_call(p2[0, :N], p2[1, :N], h2p, d0, d1,
                     b2r, g2r, be2r, rm2r, rv2r, fw, fb)
    return out[:, :C]


# speedup vs baseline: 1.9592x; 1.0006x over previous
import functools

import jax
import jax.numpy as jnp
from jax import lax
from jax.experimental import pallas as pl
from jax.experimental.pallas import tpu as pltpu
from jax.experimental.pallas import tpu_sc as plsc

N = 10000
E = 320000
D = 128
H = 128
C = 40
EPS = 1e-5

NC = 2
NS = 16
NW = NC * NS
K = 128
T = 80
E_PAD = NW * T * K
OROWS = 632
OPAD = NS * OROWS
ACC_ROWS = OPAD
ZROWS = OROWS

@functools.lru_cache(maxsize=None)
def _sc_kernels():
    mesh = plsc.VectorSubcoreMesh(core_axis_name="c", subcore_axis_name="s",
                                  num_cores=NC, num_subcores=NS)

    @functools.partial(
        pl.kernel,
        out_type=jax.ShapeDtypeStruct((NC, OPAD, H), jnp.float32),
        mesh=mesh,
        scratch_types=[
            pltpu.VMEM((T, K), jnp.int32),
            pltpu.VMEM((K, H), jnp.float32),
            pltpu.VMEM_SHARED((ACC_ROWS, H), jnp.float32),
            pltpu.SemaphoreType.DMA,
        ],
    )
    def deg_kernel(dst_hbm, ones_hbm, zeros_hbm, out_hbm, dst_all, ones_v,
                   acc, sem):
        cid = lax.axis_index("c")
        sid = lax.axis_index("s")
        wid = sid * NC + cid
        pltpu.sync_copy(zeros_hbm, acc.at[pl.ds(sid * ZROWS, ZROWS)])
        pltpu.sync_copy(ones_hbm, ones_v)
        pltpu.sync_copy(dst_hbm.at[pl.ds(wid * T, T)], dst_all)
        plsc.subcore_barrier()

        def chunk(c, carry):
            pltpu.async_copy(ones_v, acc.at[dst_all.at[c]], sem, add=True)
            return carry

        lax.fori_loop(0, T, chunk, 0)

        def drain(c, carry):
            pltpu.make_async_copy(ones_v, acc.at[dst_all.at[c]], sem).wait()
            return carry

        lax.fori_loop(0, T, drain, 0)
        plsc.subcore_barrier()
        pltpu.sync_copy(acc.at[pl.ds(sid * OROWS, OROWS)],
                        out_hbm.at[cid].at[pl.ds(sid * OROWS, OROWS)])

    NBUF = 2
    IB = 4
    G2 = T // IB

    @functools.partial(
        pl.kernel,
        out_type=jax.ShapeDtypeStruct((NC, OPAD, H), jnp.float32),
        mesh=mesh,
        scratch_types=[
            pltpu.VMEM((IB, K), jnp.int32),
            pltpu.VMEM((IB, K), jnp.int32),
            pltpu.VMEM((NBUF, K, H), jnp.float32),
            pltpu.VMEM_SHARED((ACC_ROWS, H), jnp.float32),
            pltpu.SemaphoreType.DMA((NBUF,)),
            pltpu.SemaphoreType.DMA((NBUF,)),
            pltpu.SemaphoreType.DMA((IB,)),
        ],
    )
    def agg_kernel(h_hbm, src_hbm, dst_hbm, zeros_hbm, out_hbm,
                   sidx, didx, rows, acc, gsem, ssem, isem):
        cid = lax.axis_index("c")
        sid = lax.axis_index("s")
        wid = sid * NC + cid
        base = wid * T
        pltpu.sync_copy(zeros_hbm, acc.at[pl.ds(sid * ZROWS, ZROWS)])
        for b in range(NBUF):
            pltpu.async_copy(src_hbm.at[base + b], sidx.at[b], isem.at[b])
            pltpu.async_copy(dst_hbm.at[base + b], didx.at[b], isem.at[b])
        plsc.subcore_barrier()

        def do_chunk(g, u, drain_only):
            b = u % NBUF
            c = g * IB + u
            pu = (u - NBUF) % IB

            def wait_scatter():
                pltpu.make_async_copy(rows.at[b], acc.at[didx.at[pu]],
                                      ssem.at[b]).wait()

            if u < NBUF:
                pl.when(g > 0)(wait_scatter)
            else:
                wait_scatter()
            if drain_only:
                return
            pf = (u + NBUF) % IB

            def prefetch():
                pltpu.async_copy(src_hbm.at[base + c + NBUF], sidx.at[pf],
                                 isem.at[pf])
                pltpu.async_copy(dst_hbm.at[base + c + NBUF], didx.at[pf],
                                 isem.at[pf])

            if u < NBUF:
                prefetch()
            else:
                pl.when(g < G2 - 1)(prefetch)
            pltpu.make_async_copy(src_hbm.at[base + c], sidx.at[u],
                                  isem.at[u]).wait()
            pltpu.make_async_copy(dst_hbm.at[base + c], didx.at[u],
                                  isem.at[u]).wait()
            pltpu.async_copy(h_hbm.at[sidx.at[u]], rows.at[b], gsem.at[b])
            pltpu.make_async_copy(h_hbm.at[sidx.at[u]], rows.at[b],
                                  gsem.at[b]).wait()
            pltpu.async_copy(rows.at[b], acc.at[didx.at[u]], ssem.at[b],
                             add=True)

        def group(g, carry):
            for u in range(IB):
                do_chunk(g, u, False)
            return carry

        lax.fori_loop(0, G2, group, 0)
        for u in range(NBUF, IB):
            pltpu.make_async_copy(rows.at[u % NBUF], acc.at[didx.at[u]],
                                  ssem.at[u % NBUF]).wait()
        plsc.subcore_barrier()
        pltpu.sync_copy(acc.at[pl.ds(sid * OROWS, OROWS)],
                        out_hbm.at[cid].at[pl.ds(sid * OROWS, OROWS)])

    return deg_kernel, agg_kernel


BR = 1000


def _dinv(d0_ref, d1_ref):
    deg = d0_ref[:, 0:1] + d1_ref[:, 0:1] + 1.0
    return lax.rsqrt(deg)


def _a1_body(x_ref, w_ref, d0_ref, d1_ref, o_ref):
    h = jnp.dot(x_ref[...], w_ref[...], preferred_element_type=jnp.float32)
    o_ref[...] = h * _dinv(d0_ref, d1_ref)


def _bn_relu(p0, p1, hp, dinv, b, g, beta, rm, rv):
    agg = (p0[...] + p1[...] + hp[...]) * dinv + b[...]
    y = (agg - rm[...]) * lax.rsqrt(rv[...] + EPS) * g[...] + beta[...]
    return jnp.maximum(y, 0.0)


def _b1a2_body(p0, p1, hp, d0, d1, b, g, beta, rm, rv, w2, o_ref):
    dinv = _dinv(d0, d1)
    y = _bn_relu(p0, p1, hp, dinv, b, g, beta, rm, rv)
    o_ref[...] = jnp.dot(y, w2[...], preferred_element_type=jnp.float32) * dinv


def _b2fc_body(p0, p1, hp, d0, d1, b, g, beta, rm, rv, fw, fb, o_ref):
    dinv = _dinv(d0, d1)
    y = _bn_relu(p0, p1, hp, dinv, b, g, beta, rm, rv)
    logits = jnp.dot(y, fw[...], preferred_element_type=jnp.float32) + fb[...]
    col = lax.broadcasted_iota(jnp.int32, logits.shape, 1)
    logits = jnp.where(col < C, logits, -1e30)
    m = jnp.max(logits, axis=1, keepdims=True)
    s = jnp.sum(jnp.exp(logits - m), axis=1, keepdims=True)
    o_ref[...] = logits - m - jnp.log(s)


def _row_spec(w):
    return pl.BlockSpec((BR, w), lambda i: (i, 0))


def _full_spec(r, c):
    return pl.BlockSpec((r, c), lambda i: (0, 0))


_NH = jax.ShapeDtypeStruct((N, H), jnp.float32)

_a1_call = pl.pallas_call(
    _a1_body, grid=(N // BR,),
    in_specs=[_row_spec(D), _full_spec(D, H), _row_spec(16), _row_spec(16)],
    out_specs=_row_spec(H), out_shape=_NH)

_b1a2_call = pl.pallas_call(
    _b1a2_body, grid=(N // BR,),
    in_specs=[_row_spec(H), _row_spec(H), _row_spec(H), _row_spec(16),
              _row_spec(16)] + [_full_spec(1, H)] * 5 + [_full_spec(H, H)],
    out_specs=_row_spec(H), out_shape=_NH)

_b2fc_call = pl.pallas_call(
    _b2fc_body, grid=(N // BR,),
    in_specs=[_row_spec(H), _row_spec(H), _row_spec(H), _row_spec(16),
              _row_spec(16)] + [_full_spec(1, H)] * 5
             + [_full_spec(H, H), _full_spec(1, H)],
    out_specs=_row_spec(H), out_shape=_NH)


def kernel(x, edge_index, W1, b1, g1, beta1, rm1, rv1,
           W2, b2, g2, beta2, rm2, rv2, fcW, fcb):
    src = edge_index[0].astype(jnp.int32)
    dst = edge_index[1].astype(jnp.int32)
    pad = E_PAD - E
    src_p = jnp.concatenate([src, jnp.zeros((pad,), jnp.int32)]).reshape(-1, K)
    dst_p = jnp.concatenate([dst, jnp.full((pad,), N, jnp.int32)]).reshape(-1, K)

    onesH = jnp.ones((K, H), jnp.float32)
    zerosH = jnp.zeros((ZROWS, H), jnp.float32)

    deg_kernel, agg_kernel = _sc_kernels()
    degp = deg_kernel(dst_p, onesH, zerosH)
    d0, d1 = degp[0, :N, :16], degp[1, :N, :16]

    b1r, g1r, be1r = b1.reshape(1, H), g1.reshape(1, H), beta1.reshape(1, H)
    rm1r, rv1r = rm1.reshape(1, H), rv1.reshape(1, H)
    b2r, g2r, be2r = b2.reshape(1, H), g2.reshape(1, H), beta2.reshape(1, H)
    rm2r, rv2r = rm2.reshape(1, H), rv2.reshape(1, H)
    fw = jnp.pad(fcW, ((0, 0), (0, H - C)))
    fb = jnp.pad(fcb, (0, H - C)).reshape(1, H)

    h1p = _a1_call(x, W1, d0, d1)
    p1 = agg_kernel(h1p, src_p, dst_p, zerosH)
    h2p = _b1a2_call(p1[0, :N], p1[1, :N], h1p, d0, d1,
                     b1r, g1r, be1r, rm1r, rv1r, W2)
    p2 = agg_kernel(h2p, src_p, dst_p, zerosH)
    out = _b2fc_call(p2[0, :N], p2[1, :N], h2p, d0, d1,
                     b2r, g2r, be2r, rm2r, rv2r, fw, fb)
    return out[:, :C]
